# Initial kernel scaffold; baseline (speedup 1.0000x reference)
import jax
import jax.numpy as jnp
from jax.experimental import pallas as pl


def kernel(text_feature, frame_features, segment_indices, W_gcn1, W_gcn2, gru_W_ih, gru_W_hh, gru_b_ih, gru_b_hh, h0_Wt, h0_Wf, h0_Wmf, h0_Wmt, h0_We2t, h0_We2f, h1_Wt, h1_Wf, h1_Wmf, h1_Wmt, h1_We2t, h1_We2f, W_fc, b_fc):
    raise NotImplementedError("write your pallas kernel here")


# speedup vs baseline: 1.5062x; 1.5062x over previous
import functools
import math

import jax
import jax.numpy as jnp
from jax import lax
from jax.experimental import pallas as pl
from jax.experimental.pallas import tpu as pltpu
from jax.experimental.pallas import tpu_sc as plsc

N = 4096
D_FEAT = 128
D_HID = 64
T_TOK = 77
D_TXT = 768
N_SEG = 32
SEG_LEN = 64

ROW_BLK = 512
N_BLK = N // ROW_BLK

_F32 = jnp.float32


def _k1_body(frames_ref, w1_ref, w2_ref, bi_ref, y_ref):
    xw = jax.lax.dot_general(frames_ref[...], w1_ref[...],
                             (((1,), (0,)), ((), ())),
                             preferred_element_type=_F32)
    row = lax.broadcasted_iota(jnp.int32, (N, 1), 0)
    dinv = jnp.where((row == 0) | (row == N - 1),
                     1.0 / math.sqrt(2.0), 1.0 / math.sqrt(3.0))
    s = dinv * xw
    zero = jnp.zeros((1, D_HID), _F32)
    up = jnp.concatenate([zero, s[:-1, :]], axis=0)
    dn = jnp.concatenate([s[1:, :], zero], axis=0)
    bi = jax.nn.relu(dinv * (up + s + dn))
    bi_ref[...] = bi
    y_ref[...] = jax.lax.dot_general(bi, w2_ref[...],
                                     (((1,), (0,)), ((), ())),
                                     preferred_element_type=_F32)


def _dist_tile(xb, xf):
    sqb = jnp.sum(xb * xb, axis=1, keepdims=True)
    sqf = jnp.sum(xf * xf, axis=1)[None, :]
    xxt = jax.lax.dot_general(xb, xf, (((1,), (1,)), ((), ())),
                              preferred_element_type=_F32)
    d2 = sqb + sqf - 2.0 * xxt
    return jnp.sqrt(jnp.maximum(d2, 0.0))


def _k2_body(xb_ref, xf_ref, sum_ref):
    i = pl.program_id(0)
    dist = _dist_tile(xb_ref[...], xf_ref[...])
    partial = jnp.sum(dist)

    @pl.when(i == 0)
    def _init():
        sum_ref[...] = jnp.zeros_like(sum_ref)

    sum_ref[...] = sum_ref[...] + partial


def _k3_body(xb_ref, xf_ref, y_ref, dsum_ref, wmf_ref, wf_out_ref, m0_ref):
    i = pl.program_id(0)
    stat = dsum_ref[...] * (1.0 / (N * N))
    dist = _dist_tile(xb_ref[...], xf_ref[...])
    adj = jnp.exp(dist * (-1.0 / (stat + 1e-6)))
    rowsum = jnp.sum(adj, axis=1, keepdims=True)
    acc = jax.lax.dot_general(adj, y_ref[...], (((1,), (0,)), ((), ())),
                              preferred_element_type=_F32)
    wf = jax.nn.relu(acc / (rowsum + 1e-6))
    wf_out_ref[...] = wf
    pm = jnp.sum(jax.nn.relu(
        jax.lax.dot_general(wf, wmf_ref[...], (((1,), (0,)), ((), ())),
                            preferred_element_type=_F32)), axis=0,
        keepdims=True)

    @pl.when(i == 0)
    def _init():
        m0_ref[...] = jnp.zeros_like(m0_ref)

    m0_ref[...] += pm


def _k4_body(text_ref, wih_ref, whh_ref, bih_ref, bhh_ref, t_ref, gi_ref):
    gi_ref[...] = jax.lax.dot_general(text_ref[...], wih_ref[...],
                                      (((1,), (0,)), ((), ())),
                                      preferred_element_type=_F32) + bih_ref[...]
    whh = whh_ref[...]
    bhh = bhh_ref[...]
    H = D_HID

    def step(i, h):
        gi = gi_ref[pl.ds(i, 1), :]
        gh = jax.lax.dot_general(h, whh, (((1,), (0,)), ((), ())),
                                 preferred_element_type=_F32) + bhh
        r = jax.nn.sigmoid(gi[:, :H] + gh[:, :H])
        z = jax.nn.sigmoid(gi[:, H:2 * H] + gh[:, H:2 * H])
        n = jnp.tanh(gi[:, 2 * H:] + r * gh[:, 2 * H:])
        return (1.0 - z) * n + z * h

    t_ref[...] = lax.fori_loop(0, T_TOK, step, jnp.zeros((1, H), _F32))


def _k5_body(wf_ref, t_ref, m0_ref, wt0_ref, we2t0_ref, wmt0_ref, we2f0_ref,
             wf0_ref, wmt1_ref, we2f1_ref, wf1_ref, f1_out_ref):
    def mm(a, b_ref):
        return jax.lax.dot_general(a, b_ref[...], (((1,), (0,)), ((), ())),
                                   preferred_element_type=_F32)

    t = t_ref[...]
    m0 = m0_ref[...] * (1.0 / N)
    t0 = jax.nn.relu(mm(t, wt0_ref) + mm(m0, we2t0_ref))
    bias0 = mm(jax.nn.relu(mm(t, wmt0_ref)), we2f0_ref)
    f0 = jax.nn.relu(mm(wf_ref[...], wf0_ref) + bias0)
    bias1 = mm(jax.nn.relu(mm(t0, wmt1_ref)), we2f1_ref)
    f1_out_ref[...] = jax.nn.relu(mm(f0, wf1_ref) + bias1)


def _seg_pool_sc(seg_idx, f1, w_fc_vec, b_fc):
    mesh = plsc.VectorSubcoreMesh(core_axis_name="c", subcore_axis_name="s")

    @functools.partial(
        pl.kernel, mesh=mesh,
        compiler_params=pltpu.CompilerParams(needs_layout_passes=False,
                                             use_tc_tiling_on_sc=False),
        out_type=jax.ShapeDtypeStruct((N_SEG, 16), _F32),
        scratch_types=[
            pltpu.VMEM((SEG_LEN,), jnp.int32),
            pltpu.VMEM((SEG_LEN, D_HID), _F32),
            pltpu.VMEM((D_HID,), _F32),
            pltpu.VMEM((16,), _F32),
            pltpu.VMEM((16,), _F32),
            pltpu.SemaphoreType.DMA,
        ],
    )
    def seg_kernel(idx_hbm, f_hbm, wfc_hbm, bfc_hbm, out_hbm,
                   idx_v, rows_v, wfc_v, bfc_v, res_v, sem):
        wid = lax.axis_index("s") * 2 + lax.axis_index("c")
        pltpu.sync_copy(wfc_hbm, wfc_v)
        pltpu.sync_copy(bfc_hbm, bfc_v)
        pltpu.sync_copy(idx_hbm.at[wid], idx_v)
        pltpu.async_copy(f_hbm.at[idx_v], rows_v, sem).wait()
        acc = jnp.zeros((16,), _F32)
        for r in range(SEG_LEN):
            for c in range(D_HID // 16):
                acc = acc + rows_v[r, pl.ds(c * 16, 16)] * wfc_v[pl.ds(c * 16, 16)]
        score = jnp.sum(acc) * (1.0 / SEG_LEN)
        res_v[...] = jnp.full((16,), score, _F32) + bfc_v[...]
        pltpu.sync_copy(res_v, out_hbm.at[wid])

    return seg_kernel(seg_idx, f1, w_fc_vec, b_fc)


def kernel(text_feature, frame_features, segment_indices,
           W_gcn1, W_gcn2, gru_W_ih, gru_W_hh, gru_b_ih, gru_b_hh,
           h0_Wt, h0_Wf, h0_Wmf, h0_Wmt, h0_We2t, h0_We2f,
           h1_Wt, h1_Wf, h1_Wmf, h1_Wmt, h1_We2t, h1_We2f,
           W_fc, b_fc):
    full = lambda shape: pl.BlockSpec(shape, lambda *_: tuple(0 for _ in shape))

    bi, y = pl.pallas_call(
        _k1_body,
        out_shape=[jax.ShapeDtypeStruct((N, D_HID), _F32),
                   jax.ShapeDtypeStruct((N, D_HID), _F32)],
    )(frame_features, W_gcn1, W_gcn2)

    dsum = pl.pallas_call(
        _k2_body,
        grid=(N_BLK,),
        in_specs=[pl.BlockSpec((ROW_BLK, D_HID), lambda i: (i, 0)),
                  full((N, D_HID))],
        out_specs=full((1, 1)),
        out_shape=jax.ShapeDtypeStruct((1, 1), _F32),
    )(bi, bi)

    wf, m0 = pl.pallas_call(
        _k3_body,
        grid=(N_BLK,),
        in_specs=[pl.BlockSpec((ROW_BLK, D_HID), lambda i: (i, 0)),
                  full((N, D_HID)), full((N, D_HID)), full((1, 1)),
                  full((D_HID, D_HID))],
        out_specs=[pl.BlockSpec((ROW_BLK, D_HID), lambda i: (i, 0)),
                   full((1, D_HID))],
        out_shape=[jax.ShapeDtypeStruct((N, D_HID), _F32),
                   jax.ShapeDtypeStruct((1, D_HID), _F32)],
    )(bi, bi, y, dsum, h0_Wmf)

    t = pl.pallas_call(
        _k4_body,
        out_shape=jax.ShapeDtypeStruct((1, D_HID), _F32),
        scratch_shapes=[pltpu.VMEM((T_TOK, 3 * D_HID), _F32)],
    )(text_feature, gru_W_ih, gru_W_hh,
      gru_b_ih.reshape(1, -1), gru_b_hh.reshape(1, -1))

    f1 = pl.pallas_call(
        _k5_body,
        out_shape=jax.ShapeDtypeStruct((N, D_HID), _F32),
    )(wf, t, m0, h0_Wt, h0_We2t, h0_Wmt, h0_We2f, h0_Wf,
      h1_Wmt, h1_We2f, h1_Wf)

    seg_out = _seg_pool_sc(segment_indices.astype(jnp.int32), f1,
                           W_fc.reshape(-1),
                           jnp.broadcast_to(b_fc, (16,)))
    return seg_out[:, 0]


# speedup vs baseline: 1.5795x; 1.0486x over previous
import functools
import math

import jax
import jax.numpy as jnp
from jax import lax
from jax.experimental import pallas as pl
from jax.experimental.pallas import tpu as pltpu
from jax.experimental.pallas import tpu_sc as plsc

N = 4096
D_FEAT = 128
D_HID = 64
T_TOK = 77
D_TXT = 768
N_SEG = 32
SEG_LEN = 64

ROW_BLK = 512
N_BLK = N // ROW_BLK

_F32 = jnp.float32


def _mm(a, b):
    return jax.lax.dot_general(a, b, (((1,), (0,)), ((), ())),
                               preferred_element_type=_F32)


def _mega_body(frames_ref, text_ref, w1_ref, w2_ref,
               wih_ref, whh_ref, bih_ref, bhh_ref,
               wmf_ref, wt0_ref, we2t0_ref, wmt0_ref, we2f0_ref, wf0_ref,
               wmt1_ref, we2f1_ref, wf1_ref,
               f1_out_ref,
               bi_s, y_s, wf_s, sqc_s, sqr_s, gi_s):
    xw = _mm(frames_ref[...], w1_ref[...])
    row = lax.broadcasted_iota(jnp.int32, (N, 1), 0)
    dinv = jnp.where((row == 0) | (row == N - 1),
                     1.0 / math.sqrt(2.0), 1.0 / math.sqrt(3.0))
    s = dinv * xw
    zero = jnp.zeros((1, D_HID), _F32)
    up = jnp.concatenate([zero, s[:-1, :]], axis=0)
    dn = jnp.concatenate([s[1:, :], zero], axis=0)
    bi = jax.nn.relu(dinv * (up + s + dn))
    bi_s[...] = bi
    y_s[...] = _mm(bi, w2_ref[...])
    sq = jnp.sum(bi * bi, axis=1)
    sqc_s[...] = sq[:, None]
    sqr_s[...] = sq[None, :]

    def dist_blk(i):
        off = pl.multiple_of(i * ROW_BLK, ROW_BLK)
        xb = bi_s[pl.ds(off, ROW_BLK), :]
        sqb = sqc_s[pl.ds(off, ROW_BLK), :]
        xxt = jax.lax.dot_general(xb, bi_s[...], (((1,), (1,)), ((), ())),
                                  preferred_element_type=_F32)
        d2 = sqb + sqr_s[...] - 2.0 * xxt
        return jnp.sqrt(jnp.maximum(d2, 0.0))

    def pass_a(i, dsum):
        return dsum + jnp.sum(dist_blk(i))

    dsum = lax.fori_loop(0, N_BLK, pass_a, jnp.float32(0.0))
    stat = dsum * (1.0 / (N * N))
    neg_inv = -1.0 / (stat + 1e-6)

    def pass_b(i, m0):
        off = pl.multiple_of(i * ROW_BLK, ROW_BLK)
        adj = jnp.exp(dist_blk(i) * neg_inv)
        rowsum = jnp.sum(adj, axis=1, keepdims=True)
        acc = _mm(adj, y_s[...])
        wfb = jax.nn.relu(acc / (rowsum + 1e-6))
        wf_s[pl.ds(off, ROW_BLK), :] = wfb
        return m0 + jnp.sum(jax.nn.relu(_mm(wfb, wmf_ref[...])),
                            axis=0, keepdims=True)

    m0 = lax.fori_loop(0, N_BLK, pass_b, jnp.zeros((1, D_HID), _F32))

    gi_s[...] = _mm(text_ref[...], wih_ref[...]) + bih_ref[...]
    whh = whh_ref[...]
    bhh = bhh_ref[...]
    H = D_HID

    def gru_step(i, h):
        gi = gi_s[pl.ds(i, 1), :]
        gh = _mm(h, whh) + bhh
        r = jax.nn.sigmoid(gi[:, :H] + gh[:, :H])
        z = jax.nn.sigmoid(gi[:, H:2 * H] + gh[:, H:2 * H])
        n = jnp.tanh(gi[:, 2 * H:] + r * gh[:, 2 * H:])
        return (1.0 - z) * n + z * h

    t = lax.fori_loop(0, T_TOK, gru_step, jnp.zeros((1, H), _F32))

    m0 = m0 * (1.0 / N)
    t0 = jax.nn.relu(_mm(t, wt0_ref[...]) + _mm(m0, we2t0_ref[...]))
    bias0 = _mm(jax.nn.relu(_mm(t, wmt0_ref[...])), we2f0_ref[...])
    f0 = jax.nn.relu(_mm(wf_s[...], wf0_ref[...]) + bias0)
    bias1 = _mm(jax.nn.relu(_mm(t0, wmt1_ref[...])), we2f1_ref[...])
    f1_out_ref[...] = jax.nn.relu(_mm(f0, wf1_ref[...]) + bias1)


def _seg_pool_sc(seg_idx, f1, w_fc_vec, b_fc16):
    mesh = plsc.VectorSubcoreMesh(core_axis_name="c", subcore_axis_name="s")

    @functools.partial(
        pl.kernel, mesh=mesh,
        compiler_params=pltpu.CompilerParams(needs_layout_passes=False,
                                             use_tc_tiling_on_sc=False),
        out_type=jax.ShapeDtypeStruct((N_SEG, 16), _F32),
        scratch_types=[
            pltpu.VMEM((SEG_LEN,), jnp.int32),
            pltpu.VMEM((SEG_LEN, D_HID), _F32),
            pltpu.VMEM((D_HID,), _F32),
            pltpu.VMEM((16,), _F32),
            pltpu.VMEM((16,), _F32),
            pltpu.SemaphoreType.DMA,
        ],
    )
    def seg_kernel(idx_hbm, f_hbm, wfc_hbm, bfc_hbm, out_hbm,
                   idx_v, rows_v, wfc_v, bfc_v, res_v, sem):
        wid = lax.axis_index("s") * 2 + lax.axis_index("c")
        pltpu.sync_copy(wfc_hbm, wfc_v)
        pltpu.sync_copy(bfc_hbm, bfc_v)
        pltpu.sync_copy(idx_hbm.at[wid], idx_v)
        pltpu.async_copy(f_hbm.at[idx_v], rows_v, sem).wait()
        acc = jnp.zeros((16,), _F32)
        for r in range(SEG_LEN):
            for c in range(D_HID // 16):
                acc = acc + rows_v[r, pl.ds(c * 16, 16)] * wfc_v[pl.ds(c * 16, 16)]
        score = jnp.sum(acc) * (1.0 / SEG_LEN)
        res_v[...] = jnp.full((16,), score, _F32) + bfc_v[...]
        pltpu.sync_copy(res_v, out_hbm.at[wid])

    return seg_kernel(seg_idx, f1, w_fc_vec, b_fc16)


def kernel(text_feature, frame_features, segment_indices,
           W_gcn1, W_gcn2, gru_W_ih, gru_W_hh, gru_b_ih, gru_b_hh,
           h0_Wt, h0_Wf, h0_Wmf, h0_Wmt, h0_We2t, h0_We2f,
           h1_Wt, h1_Wf, h1_Wmf, h1_Wmt, h1_We2t, h1_We2f,
           W_fc, b_fc):
    f1 = pl.pallas_call(
        _mega_body,
        out_shape=jax.ShapeDtypeStruct((N, D_HID), _F32),
        scratch_shapes=[
            pltpu.VMEM((N, D_HID), _F32),
            pltpu.VMEM((N, D_HID), _F32),
            pltpu.VMEM((N, D_HID), _F32),
            pltpu.VMEM((N, 1), _F32),
            pltpu.VMEM((1, N), _F32),
            pltpu.VMEM((T_TOK, 3 * D_HID), _F32),
        ],
    )(frame_features, text_feature, W_gcn1, W_gcn2,
      gru_W_ih, gru_W_hh, gru_b_ih.reshape(1, -1), gru_b_hh.reshape(1, -1),
      h0_Wmf, h0_Wt, h0_We2t, h0_Wmt, h0_We2f, h0_Wf,
      h1_Wmt, h1_We2f, h1_Wf)

    seg_out = _seg_pool_sc(segment_indices.astype(jnp.int32), f1,
                           W_fc.reshape(-1),
                           jnp.broadcast_to(b_fc, (16,)))
    return seg_out[:, 0]
